# Initial kernel scaffold; baseline (speedup 1.0000x reference)
import jax
import jax.numpy as jnp
from jax.experimental import pallas as pl


def kernel(hour, weekday, time_sin_cos, hour_table, weekday_table, W1, b1, gamma, beta, W2, b2):
    raise NotImplementedError("write your pallas kernel here")


# speedup vs baseline: 3.5148x; 3.5148x over previous
import functools
import math

import jax
import jax.numpy as jnp
from jax.experimental import pallas as pl


def _body(hour_ref, wday_ref, tsc_ref, tbl_ref, w0_ref, w1_ref, b1_ref,
          gamma_ref, beta_ref, w2t_ref, b2_ref, out_ref):
    h = hour_ref[0]
    w = wday_ref[0]
    bb = h.shape[0]
    iota = jax.lax.broadcasted_iota(jnp.int32, (bb, 32), 1)
    oh = jnp.logical_or(iota == h, iota == (w + 24)).astype(jnp.float32)
    emb = jnp.dot(oh, tbl_ref[...], preferred_element_type=jnp.float32)

    tsc = tsc_ref[0]
    s0 = tsc[:, 0:1]
    s1 = tsc[:, 1:2]
    t = s0 * w0_ref[...] + s1 * w1_ref[...] + b1_ref[...]
    mean = jnp.mean(t, axis=1, keepdims=True)
    var = jnp.mean((t - mean) ** 2, axis=1, keepdims=True)
    t = (t - mean) * jax.lax.rsqrt(var + 1e-5) * gamma_ref[...] + beta_ref[...]
    t = 0.5 * t * (1.0 + jax.lax.erf(t * (1.0 / math.sqrt(2.0))))
    t2 = jnp.dot(t, w2t_ref[...], preferred_element_type=jnp.float32) + b2_ref[...]

    out_ref[:, 0:64] = emb
    out_ref[:, 64:128] = t2


@functools.partial(jax.jit, static_argnames=())
def kernel(hour, weekday, time_sin_cos, hour_table, weekday_table,
           W1, b1, gamma, beta, W2, b2):
    B, L = hour.shape
    D4 = hour_table.shape[1]
    D2 = 2 * D4
    rows = B * L
    bb = next(c for c in (8192, 6400, 4096, 3200, 2048, 1600, 1024, 800,
                          512, 400, 256, 128, 64, 32, 16, 8)
              if rows % c == 0)
    grid = rows // bb

    tbl = jnp.zeros((32, D2), jnp.float32)
    tbl = tbl.at[0:24, 0:D4].set(hour_table)
    tbl = tbl.at[24:31, D4:D2].set(weekday_table)

    hour_r = hour.reshape(grid, bb, 1).astype(jnp.int32)
    wday_r = weekday.reshape(grid, bb, 1).astype(jnp.int32)
    tsc_r = time_sin_cos.reshape(grid, bb, 2)
    w0 = W1[:, 0].reshape(1, D2)
    w1 = W1[:, 1].reshape(1, D2)
    w2t = W2.T
    b1r = b1.reshape(1, D2)
    gammar = gamma.reshape(1, D2)
    betar = beta.reshape(1, D2)
    b2r = b2.reshape(1, D2)

    out = pl.pallas_call(
        _body,
        grid=(grid,),
        in_specs=[
            pl.BlockSpec((1, bb, 1), lambda i: (i, 0, 0)),
            pl.BlockSpec((1, bb, 1), lambda i: (i, 0, 0)),
            pl.BlockSpec((1, bb, 2), lambda i: (i, 0, 0)),
            pl.BlockSpec((32, D2), lambda i: (0, 0)),
            pl.BlockSpec((1, D2), lambda i: (0, 0)),
            pl.BlockSpec((1, D2), lambda i: (0, 0)),
            pl.BlockSpec((1, D2), lambda i: (0, 0)),
            pl.BlockSpec((1, D2), lambda i: (0, 0)),
            pl.BlockSpec((1, D2), lambda i: (0, 0)),
            pl.BlockSpec((D2, D2), lambda i: (0, 0)),
            pl.BlockSpec((1, D2), lambda i: (0, 0)),
        ],
        out_specs=pl.BlockSpec((bb, 2 * D2), lambda i: (i, 0)),
        out_shape=jax.ShapeDtypeStruct((rows, 2 * D2), jnp.float32),
    )(hour_r, wday_r, tsc_r, tbl, w0, w1, b1r, gammar, betar, w2t, b2r)

    return out.reshape(B, L, 2 * D2)
